# baseline (speedup 1.0000x reference)
import functools

import jax
import jax.numpy as jnp
from jax import lax
from jax.experimental import pallas as pl
from jax.experimental.pallas import tpu as pltpu
from jax.experimental.pallas import tpu_sc as plsc

CHUNK = 128


@functools.lru_cache(maxsize=None)
def _build(n_flat: int, emb: int, nc: int, ns: int):
    nw = nc * ns
    assert n_flat % (nw * CHUNK) == 0
    nchunks = n_flat // (nw * CHUNK)
    per_w = nchunks * CHUNK

    mesh = plsc.VectorSubcoreMesh(core_axis_name="c", subcore_axis_name="s")

    @functools.partial(
        pl.kernel,
        out_type=jax.ShapeDtypeStruct((n_flat, emb), jnp.float32),
        mesh=mesh,
        scratch_types=[
            pltpu.VMEM((nchunks, CHUNK), jnp.int32),
            pltpu.VMEM((CHUNK, emb), jnp.float32),
            pltpu.SemaphoreType.DMA,
        ],
        compiler_params=pltpu.CompilerParams(use_tc_tiling_on_sc=False),
    )
    def emb_kernel(table_hbm, idx_hbm, out_hbm, idx_v, rows_v, sem):
        wid = lax.axis_index("s") * nc + lax.axis_index("c")
        out_base = wid * per_w
        pltpu.sync_copy(idx_hbm.at[wid], idx_v)

        def body(j, carry):
            pltpu.async_copy(table_hbm.at[idx_v.at[j]], rows_v, sem).wait()
            pltpu.sync_copy(
                rows_v, out_hbm.at[pl.ds(out_base + j * CHUNK, CHUNK)]
            )
            return carry

        lax.fori_loop(0, nchunks, body, 0)

    return emb_kernel, nw


def kernel(indices, table):
    n_flat = indices.size
    emb = table.shape[1]
    info = plsc.get_sparse_core_info()
    emb_kernel, nw = _build(n_flat, emb, info.num_cores, info.num_subcores)
    idx3 = indices.reshape(nw, n_flat // (nw * CHUNK), CHUNK)
    out = emb_kernel(table, idx3)
    return out.reshape(*indices.shape, emb)


# speedup vs baseline: 1.1127x; 1.1127x over previous
import functools

import jax
import jax.numpy as jnp
from jax import lax
from jax.experimental import pallas as pl
from jax.experimental.pallas import tpu as pltpu
from jax.experimental.pallas import tpu_sc as plsc

CHUNK = 128
K = 5


@functools.lru_cache(maxsize=None)
def _build(n_flat: int, emb: int, nc: int, ns: int):
    nw = nc * ns
    assert n_flat % (nw * CHUNK) == 0
    nchunks = n_flat // (nw * CHUNK)
    per_w = nchunks * CHUNK
    assert nchunks % (2 * K) == 0
    ngroups = nchunks // K
    npairs = (ngroups - 2) // 2

    mesh = plsc.VectorSubcoreMesh(core_axis_name="c", subcore_axis_name="s")

    @functools.partial(
        pl.kernel,
        out_type=jax.ShapeDtypeStruct((n_flat, emb), jnp.float32),
        mesh=mesh,
        scratch_types=[
            pltpu.VMEM((nchunks, CHUNK), jnp.int32),
            pltpu.VMEM((2 * K, CHUNK, emb), jnp.float32),
            pltpu.SemaphoreType.DMA,
            pltpu.SemaphoreType.DMA,
            pltpu.SemaphoreType.DMA,
            pltpu.SemaphoreType.DMA,
        ],
        compiler_params=pltpu.CompilerParams(use_tc_tiling_on_sc=False),
    )
    def emb_kernel(table_hbm, idx_hbm, out_hbm, idx_v, rows_v,
                   gsem0, gsem1, osem0, osem1):
        wid = lax.axis_index("s") * nc + lax.axis_index("c")
        out_base = wid * per_w
        pltpu.sync_copy(idx_hbm.at[wid], idx_v)
        gsem = (gsem0, gsem1)
        osem = (osem0, osem1)

        def fire_gathers(g, h):
            for b in range(K):
                pltpu.async_copy(
                    table_hbm.at[idx_v.at[g * K + b]],
                    rows_v.at[h * K + b], gsem[h])

        def wait_gathers(h):
            for b in range(K):
                pltpu.make_async_copy(
                    table_hbm.at[idx_v.at[0]],
                    rows_v.at[h * K + b], gsem[h]).wait()

        def fire_outs(g, h):
            for b in range(K):
                pltpu.async_copy(
                    rows_v.at[h * K + b],
                    out_hbm.at[pl.ds(out_base + (g * K + b) * CHUNK, CHUNK)],
                    osem[h])

        def wait_outs(h):
            for b in range(K):
                pltpu.make_async_copy(
                    rows_v.at[h * K + b],
                    out_hbm.at[pl.ds(out_base, CHUNK)], osem[h]).wait()

        fire_gathers(0, 0)
        wait_gathers(0)
        fire_outs(0, 0)
        fire_gathers(1, 1)

        def pair_body(t, carry):
            g1 = 2 * t + 1
            wait_gathers(1)
            fire_outs(g1, 1)
            wait_outs(0)
            fire_gathers(g1 + 1, 0)
            wait_gathers(0)
            fire_outs(g1 + 1, 0)
            wait_outs(1)
            fire_gathers(g1 + 2, 1)
            return carry

        lax.fori_loop(0, npairs, pair_body, 0)

        wait_gathers(1)
        fire_outs(ngroups - 1, 1)
        wait_outs(0)
        wait_outs(1)

    return emb_kernel, nw


def kernel(indices, table):
    n_flat = indices.size
    emb = table.shape[1]
    info = plsc.get_sparse_core_info()
    emb_kernel, nw = _build(n_flat, emb, info.num_cores, info.num_subcores)
    idx3 = indices.reshape(nw, n_flat // (nw * CHUNK), CHUNK)
    out = emb_kernel(table, idx3)
    return out.reshape(*indices.shape, emb)
